# baseline (speedup 1.0000x reference)
import functools

import jax
import jax.numpy as jnp
from jax import lax
from jax.experimental import pallas as pl
from jax.experimental.pallas import tpu as pltpu
from jax.experimental.pallas import tpu_sc as plsc

F32 = jnp.float32
I32 = jnp.int32

_B = 2
_R = 32768
_S = 128
_NS = 128
_EPS = 1e-5
_NW = 32
_CHUNK = 32
_L = 16


def _sc_zall_body(z_hbm, w_hbm, eps_hbm, out_hbm,
                  z_v, w_v, out_v, eps_v, binsb, cdfb, hist, histA,
                  binsb2, cdfb2, hist2, histA2,
                  sin0, sin1, sout0, sout1):
    wid = lax.axis_index("s") * 2 + lax.axis_index("c")
    rays_per_w = _R // _NW
    n_chunks = rays_per_w // _CHUNK
    sin = (sin0, sin1)
    sout = (sout0, sout1)

    pltpu.sync_copy(eps_hbm, eps_v)

    iota = lax.iota(I32, _L)
    ones_i = jnp.ones((_L,), I32)
    zero_i = jnp.zeros((_L,), I32)
    n_vec = _S // _L
    for jj in range(9):
        hist[pl.ds(_L * jj, _L)] = zero_i
        histA[pl.ds(_L * jj, _L)] = zero_i
        hist2[pl.ds(_L * jj, _L)] = zero_i
        histA2[pl.ds(_L * jj, _L)] = zero_i

    def in_triple(ci, s):
        r0 = wid * rays_per_w + ci * _CHUNK
        return ((z_hbm.at[pl.ds(r0, _CHUNK), :], z_v.at[s]),
                (w_hbm.at[0, pl.ds(r0, _CHUNK), :], w_v.at[s, 0]),
                (w_hbm.at[1, pl.ds(r0, _CHUNK), :], w_v.at[s, 1]))

    def out_pair(ci, s):
        r0 = wid * rays_per_w + ci * _CHUNK
        return ((out_v.at[s, 0], out_hbm.at[0, pl.ds(r0, _CHUNK), :]),
                (out_v.at[s, 1], out_hbm.at[1, pl.ds(r0, _CHUNK), :]))

    def issue_in(ci, s):
        for src, dst in in_triple(ci, s):
            pltpu.async_copy(src, dst, sin[s])

    def wait_in(ci, s):
        for src, dst in in_triple(ci, s):
            pltpu.make_async_copy(src, dst, sin[s]).wait()

    def issue_out(ci, s):
        for src, dst in out_pair(ci, s):
            pltpu.async_copy(src, dst, sout[s])

    def wait_out(ci, s):
        for src, dst in out_pair(ci, s):
            pltpu.make_async_copy(src, dst, sout[s]).wait()

    def chunk_compute(ci, s):
        svec = jnp.full((_L,), s, I32)

        def do_ray(c, binsb, cdfb, hist, histA):
            cvec = jnp.full((_L,), 0, I32) + c
            for j in range(n_vec):
                z0 = z_v[s, c, pl.ds(_L * j, _L)]
                idx1 = iota + (_L * j + 1)
                if j == n_vec - 1:
                    idx1 = jnp.minimum(idx1, 127)
                z1 = plsc.load_gather(z_v, [svec, cvec, idx1])
                binsb[pl.ds(_L * j, _L)] = (z0 + z1) * F32(0.5)
            eps = eps_v[...]

            for b in range(_B):
                bvec = jnp.full((_L,), b, I32)
                cs_l, t_l = [], []
                for j in range(n_vec):
                    a = w_v[s, b, c, pl.ds(_L * j, _L)] + eps
                    if j == 0:
                        a = jnp.where(iota != 0, a, F32(0.0))
                    if j == n_vec - 1:
                        a = jnp.where(iota != _L - 1, a, F32(0.0))
                    cs = plsc.cumsum(a)
                    cs_l.append(cs)
                    t_l.append(jnp.max(cs))
                off = F32(0.0)
                offs = []
                for j in range(n_vec):
                    offs.append(off)
                    off = off + t_l[j]
                total = off
                rT127 = jnp.full((_L,), F32(127.0)) / total
                thrv = jnp.full((_L,), F32(1e-5)) * total
                for j in range(n_vec):
                    cv = cs_l[j] + offs[j]
                    cdfb[pl.ds(_L * j, _L)] = cv
                    m = cv * rT127
                    ti = m.astype(I32)
                    ti = jnp.where(ti.astype(F32) < m, ti + 1, ti)
                    cp = jnp.minimum(ti, 128)
                    if j == 0:
                        cp = jnp.where(iota != 0, cp, 128)
                    if j == n_vec - 1:
                        cp = jnp.where(iota != _L - 1, cp, 128)
                    plsc.addupdate_scatter(hist, [cp], ones_i)
                ih_l, it_l = [], []
                for j in range(n_vec):
                    hv = hist[pl.ds(_L * j, _L)]
                    hist[pl.ds(_L * j, _L)] = zero_i
                    csh = plsc.cumsum(hv)
                    ih_l.append(csh)
                    it_l.append(jnp.max(csh))
                ioff = 0
                ioffs = []
                for j in range(n_vec):
                    ioffs.append(ioff)
                    ioff = ioff + it_l[j]
                for j in range(n_vec):
                    ik = ih_l[j] + ioffs[j]
                    i1 = ik + 1
                    g0 = plsc.load_gather(cdfb, [ik])
                    g1 = plsc.load_gather(cdfb, [i1])
                    b0 = plsc.load_gather(binsb, [ik])
                    b1 = plsc.load_gather(binsb, [i1])
                    den = g1 - g0
                    den = jnp.where(den < thrv, F32(1.0), den)
                    u = (iota + (_L * j)).astype(F32) * F32(1.0 / 127.0)
                    sv = b0 + (u * total - g0) / den * (b1 - b0)
                    zi1 = plsc.load_gather(z_v, [svec, cvec, i1])
                    av = i1 + jnp.where(sv >= zi1, 1, 0)
                    plsc.addupdate_scatter(histA, [av], ones_i)
                    plsc.store_scatter(
                        out_v, [svec, bvec, cvec, iota + (_L * j) + av], sv)
                bh_l, bt_l = [], []
                for j in range(n_vec):
                    hv = histA[pl.ds(_L * j, _L)]
                    histA[pl.ds(_L * j, _L)] = zero_i
                    csb = plsc.cumsum(hv)
                    bh_l.append(csb)
                    bt_l.append(jnp.max(csb))
                boff = 0
                boffs = []
                for j in range(n_vec):
                    boffs.append(boff)
                    boff = boff + bt_l[j]
                for j in range(n_vec):
                    bk = bh_l[j] + boffs[j]
                    zc = z_v[s, c, pl.ds(_L * j, _L)]
                    plsc.store_scatter(
                        out_v, [svec, bvec, cvec, iota + (_L * j) + bk], zc)

        def ray_pair(cp2, _):
            do_ray(cp2 * 2, binsb, cdfb, hist, histA)
            do_ray(cp2 * 2 + 1, binsb2, cdfb2, hist2, histA2)
            return 0

        lax.fori_loop(0, _CHUNK // 2, ray_pair, 0)

    issue_in(0, 0)

    def outer_body(i2, _):
        for s in range(2):
            ci = i2 * 2 + s
            wait_in(ci, s)

            @pl.when(ci + 1 < n_chunks)
            def _prefetch():
                issue_in(ci + 1, 1 - s)

            @pl.when(ci >= 2)
            def _drain():
                wait_out(ci - 2, s)

            chunk_compute(ci, s)
            issue_out(ci, s)
        return 0

    lax.fori_loop(0, n_chunks // 2, outer_body, 0)
    wait_out(n_chunks - 2, 0)
    wait_out(n_chunks - 1, 1)


def _sc_zall(z_vals, weights, eps_vec):
    mesh = plsc.VectorSubcoreMesh(core_axis_name="c", subcore_axis_name="s")
    f = pl.kernel(
        _sc_zall_body,
        out_type=jax.ShapeDtypeStruct((_B, _R, 2 * _NS), F32),
        mesh=mesh,
        compiler_params=pltpu.CompilerParams(needs_layout_passes=False),
        scratch_types=[
            pltpu.VMEM((2, _CHUNK, _S), F32),
            pltpu.VMEM((2, _B, _CHUNK, _S), F32),
            pltpu.VMEM((2, _B, _CHUNK, 2 * _NS), F32),
            pltpu.VMEM((_L,), F32),
            pltpu.VMEM((144,), F32),
            pltpu.VMEM((144,), F32),
            pltpu.VMEM((144,), I32),
            pltpu.VMEM((144,), I32),
            pltpu.VMEM((144,), F32),
            pltpu.VMEM((144,), F32),
            pltpu.VMEM((144,), I32),
            pltpu.VMEM((144,), I32),
            pltpu.SemaphoreType.DMA,
            pltpu.SemaphoreType.DMA,
            pltpu.SemaphoreType.DMA,
            pltpu.SemaphoreType.DMA,
        ],
    )
    return f(z_vals, weights, eps_vec)


_RBLK = 512


def _tc_pts_body(z_ref, o_ref, d_ref, out_ref):
    z = z_ref[0]
    o = o_ref[...]
    d = d_ref[...]
    for c in range(3):
        out_ref[0, c] = o[:, c:c + 1] + d[:, c:c + 1] * z


def _tc_pts(z_all, rays_o_f, rays_d_f):
    nrb = _R // _RBLK
    grid = (_B, nrb)
    out4 = pl.pallas_call(
        _tc_pts_body,
        out_shape=jax.ShapeDtypeStruct((_B, 3, _R, 2 * _NS), F32),
        grid=grid,
        in_specs=[
            pl.BlockSpec((1, _RBLK, 2 * _NS), lambda b, rb: (b, rb, 0)),
            pl.BlockSpec((_RBLK, 3), lambda b, rb: (b * nrb + rb, 0)),
            pl.BlockSpec((_RBLK, 3), lambda b, rb: (b * nrb + rb, 0)),
        ],
        out_specs=pl.BlockSpec((1, 3, _RBLK, 2 * _NS),
                               lambda b, rb: (b, 0, rb, 0)),
    )(z_all, rays_o_f, rays_d_f)
    return jnp.transpose(out4, (0, 2, 3, 1))


def kernel(rays_d, rays_o, z_vals, weights, is_deterministic):
    z_vals = z_vals.reshape(-1, z_vals.shape[-1])
    eps_vec = jnp.full((_L,), _EPS, F32) * is_deterministic.astype(F32)
    z_all = _sc_zall(z_vals, weights, eps_vec)
    pts = _tc_pts(z_all, rays_o.reshape(_B * _R, 3), rays_d.reshape(_B * _R, 3))
    return pts, z_all


# speedup vs baseline: 2.1367x; 2.1367x over previous
import functools

import jax
import jax.numpy as jnp
from jax import lax
from jax.experimental import pallas as pl
from jax.experimental.pallas import tpu as pltpu
from jax.experimental.pallas import tpu_sc as plsc

F32 = jnp.float32
I32 = jnp.int32

_B = 2
_R = 32768
_S = 128
_NS = 128
_EPS = 1e-5
_NW = 32
_CHUNK = 32
_L = 16


def _sc_zall_body(z_hbm, w_hbm, eps_hbm, out_hbm,
                  z_v, w_v, out_v, eps_v, binsb, cdfb, hist, histA,
                  sin0, sin1, sout0, sout1):
    wid = lax.axis_index("s") * 2 + lax.axis_index("c")
    rays_per_w = _R // _NW
    n_chunks = rays_per_w // _CHUNK
    sin = (sin0, sin1)
    sout = (sout0, sout1)

    pltpu.sync_copy(eps_hbm, eps_v)

    iota = lax.iota(I32, _L)
    ones_i = jnp.ones((_L,), I32)
    zero_i = jnp.zeros((_L,), I32)
    n_vec = _S // _L
    for jj in range(9):
        hist[pl.ds(_L * jj, _L)] = zero_i
        histA[pl.ds(_L * jj, _L)] = zero_i

    def in_triple(ci, s):
        r0 = wid * rays_per_w + ci * _CHUNK
        return ((z_hbm.at[pl.ds(r0, _CHUNK), :], z_v.at[s]),
                (w_hbm.at[0, pl.ds(r0, _CHUNK), :], w_v.at[s, 0]),
                (w_hbm.at[1, pl.ds(r0, _CHUNK), :], w_v.at[s, 1]))

    def out_pair(ci, s):
        r0 = wid * rays_per_w + ci * _CHUNK
        return ((out_v.at[s, 0], out_hbm.at[0, pl.ds(r0, _CHUNK), :]),
                (out_v.at[s, 1], out_hbm.at[1, pl.ds(r0, _CHUNK), :]))

    def issue_in(ci, s):
        for src, dst in in_triple(ci, s):
            pltpu.async_copy(src, dst, sin[s])

    def wait_in(ci, s):
        for src, dst in in_triple(ci, s):
            pltpu.make_async_copy(src, dst, sin[s]).wait()

    def issue_out(ci, s):
        for src, dst in out_pair(ci, s):
            pltpu.async_copy(src, dst, sout[s])

    def wait_out(ci, s):
        for src, dst in out_pair(ci, s):
            pltpu.make_async_copy(src, dst, sout[s]).wait()

    def chunk_compute(ci, s):
        svec = jnp.full((_L,), s, I32)

        def do_ray(c, binsb, cdfb, hist, histA):
            cvec = jnp.full((_L,), 0, I32) + c
            for j in range(n_vec):
                z0 = z_v[s, c, pl.ds(_L * j, _L)]
                idx1 = iota + (_L * j + 1)
                if j == n_vec - 1:
                    idx1 = jnp.minimum(idx1, 127)
                z1 = plsc.load_gather(z_v, [svec, cvec, idx1])
                binsb[pl.ds(_L * j, _L)] = (z0 + z1) * F32(0.5)
            eps = eps_v[...]

            for b in range(_B):
                bvec = jnp.full((_L,), b, I32)
                cs_l, t_l = [], []
                for j in range(n_vec):
                    a = w_v[s, b, c, pl.ds(_L * j, _L)] + eps
                    if j == 0:
                        a = jnp.where(iota != 0, a, F32(0.0))
                    if j == n_vec - 1:
                        a = jnp.where(iota != _L - 1, a, F32(0.0))
                    cs = plsc.cumsum(a)
                    cs_l.append(cs)
                    t_l.append(jnp.max(cs))
                off = F32(0.0)
                offs = []
                for j in range(n_vec):
                    offs.append(off)
                    off = off + t_l[j]
                total = off
                rT127 = jnp.full((_L,), F32(127.0)) / total
                thrv = jnp.full((_L,), F32(1e-5)) * total
                for j in range(n_vec):
                    cv = cs_l[j] + offs[j]
                    cdfb[pl.ds(_L * j, _L)] = cv
                    m = cv * rT127
                    ti = m.astype(I32)
                    ti = jnp.where(ti.astype(F32) < m, ti + 1, ti)
                    cp = jnp.minimum(ti, 128)
                    if j == 0:
                        cp = jnp.where(iota != 0, cp, 128)
                    if j == n_vec - 1:
                        cp = jnp.where(iota != _L - 1, cp, 128)
                    plsc.addupdate_scatter(hist, [cp], ones_i)
                ih_l, it_l = [], []
                for j in range(n_vec):
                    hv = hist[pl.ds(_L * j, _L)]
                    hist[pl.ds(_L * j, _L)] = zero_i
                    csh = plsc.cumsum(hv)
                    ih_l.append(csh)
                    it_l.append(jnp.max(csh))
                ioff = 0
                ioffs = []
                for j in range(n_vec):
                    ioffs.append(ioff)
                    ioff = ioff + it_l[j]
                for j in range(n_vec):
                    ik = ih_l[j] + ioffs[j]
                    i1 = ik + 1
                    g0 = plsc.load_gather(cdfb, [ik])
                    g1 = plsc.load_gather(cdfb, [i1])
                    b0 = plsc.load_gather(binsb, [ik])
                    b1 = plsc.load_gather(binsb, [i1])
                    den = g1 - g0
                    den = jnp.where(den < thrv, F32(1.0), den)
                    u = (iota + (_L * j)).astype(F32) * F32(1.0 / 127.0)
                    sv = b0 + (u * total - g0) / den * (b1 - b0)
                    zi1 = plsc.load_gather(z_v, [svec, cvec, i1])
                    av = i1 + jnp.where(sv >= zi1, 1, 0)
                    plsc.addupdate_scatter(histA, [av], ones_i)
                    plsc.store_scatter(
                        out_v, [svec, bvec, cvec, iota + (_L * j) + av], sv)
                bh_l, bt_l = [], []
                for j in range(n_vec):
                    hv = histA[pl.ds(_L * j, _L)]
                    histA[pl.ds(_L * j, _L)] = zero_i
                    csb = plsc.cumsum(hv)
                    bh_l.append(csb)
                    bt_l.append(jnp.max(csb))
                boff = 0
                boffs = []
                for j in range(n_vec):
                    boffs.append(boff)
                    boff = boff + bt_l[j]
                for j in range(n_vec):
                    bk = bh_l[j] + boffs[j]
                    zc = z_v[s, c, pl.ds(_L * j, _L)]
                    plsc.store_scatter(
                        out_v, [svec, bvec, cvec, iota + (_L * j) + bk], zc)

        def ray_body(c, _):
            do_ray(c, binsb, cdfb, hist, histA)
            return 0

        lax.fori_loop(0, _CHUNK, ray_body, 0)

    issue_in(0, 0)

    def outer_body(i2, _):
        for s in range(2):
            ci = i2 * 2 + s
            wait_in(ci, s)

            @pl.when(ci + 1 < n_chunks)
            def _prefetch():
                issue_in(ci + 1, 1 - s)

            @pl.when(ci >= 2)
            def _drain():
                wait_out(ci - 2, s)

            chunk_compute(ci, s)
            issue_out(ci, s)
        return 0

    lax.fori_loop(0, n_chunks // 2, outer_body, 0)
    wait_out(n_chunks - 2, 0)
    wait_out(n_chunks - 1, 1)


def _sc_zall(z_vals, weights, eps_vec):
    mesh = plsc.VectorSubcoreMesh(core_axis_name="c", subcore_axis_name="s")
    f = pl.kernel(
        _sc_zall_body,
        out_type=jax.ShapeDtypeStruct((_B, _R, 2 * _NS), F32),
        mesh=mesh,
        compiler_params=pltpu.CompilerParams(needs_layout_passes=False),
        scratch_types=[
            pltpu.VMEM((2, _CHUNK, _S), F32),
            pltpu.VMEM((2, _B, _CHUNK, _S), F32),
            pltpu.VMEM((2, _B, _CHUNK, 2 * _NS), F32),
            pltpu.VMEM((_L,), F32),
            pltpu.VMEM((144,), F32),
            pltpu.VMEM((144,), F32),
            pltpu.VMEM((144,), I32),
            pltpu.VMEM((144,), I32),
            pltpu.SemaphoreType.DMA,
            pltpu.SemaphoreType.DMA,
            pltpu.SemaphoreType.DMA,
            pltpu.SemaphoreType.DMA,
        ],
    )
    return f(z_vals, weights, eps_vec)


_RBLK = 512


def _tc_pts_body(z_ref, o_ref, d_ref, out_ref):
    z = z_ref[0]
    o = o_ref[...]
    d = d_ref[...]
    for c in range(3):
        out_ref[0, c] = o[:, c:c + 1] + d[:, c:c + 1] * z


def _tc_pts(z_all, rays_o_f, rays_d_f):
    nrb = _R // _RBLK
    grid = (_B, nrb)
    out4 = pl.pallas_call(
        _tc_pts_body,
        out_shape=jax.ShapeDtypeStruct((_B, 3, _R, 2 * _NS), F32),
        grid=grid,
        in_specs=[
            pl.BlockSpec((1, _RBLK, 2 * _NS), lambda b, rb: (b, rb, 0)),
            pl.BlockSpec((_RBLK, 3), lambda b, rb: (b * nrb + rb, 0)),
            pl.BlockSpec((_RBLK, 3), lambda b, rb: (b * nrb + rb, 0)),
        ],
        out_specs=pl.BlockSpec((1, 3, _RBLK, 2 * _NS),
                               lambda b, rb: (b, 0, rb, 0)),
    )(z_all, rays_o_f, rays_d_f)
    return jnp.transpose(out4, (0, 2, 3, 1))


def kernel(rays_d, rays_o, z_vals, weights, is_deterministic):
    z_vals = z_vals.reshape(-1, z_vals.shape[-1])
    eps_vec = jnp.full((_L,), _EPS, F32) * is_deterministic.astype(F32)
    z_all = _sc_zall(z_vals, weights, eps_vec)
    pts = _tc_pts(z_all, rays_o.reshape(_B * _R, 3), rays_d.reshape(_B * _R, 3))
    return pts, z_all


# speedup vs baseline: 2.1408x; 1.0019x over previous
import functools

import jax
import jax.numpy as jnp
from jax import lax
from jax.experimental import pallas as pl
from jax.experimental.pallas import tpu as pltpu
from jax.experimental.pallas import tpu_sc as plsc

F32 = jnp.float32
I32 = jnp.int32

_B = 2
_R = 32768
_S = 128
_NS = 128
_EPS = 1e-5
_NW = 32
_CHUNK = 64
_L = 16


def _sc_zall_body(z_hbm, w_hbm, eps_hbm, out_hbm,
                  z_v, w_v, out_v, eps_v, binsb, cdfb, hist, histA,
                  sin0, sin1, sout0, sout1):
    wid = lax.axis_index("s") * 2 + lax.axis_index("c")
    rays_per_w = _R // _NW
    n_chunks = rays_per_w // _CHUNK
    sin = (sin0, sin1)
    sout = (sout0, sout1)

    pltpu.sync_copy(eps_hbm, eps_v)

    iota = lax.iota(I32, _L)
    ones_i = jnp.ones((_L,), I32)
    zero_i = jnp.zeros((_L,), I32)
    n_vec = _S // _L
    for jj in range(9):
        hist[pl.ds(_L * jj, _L)] = zero_i
        histA[pl.ds(_L * jj, _L)] = zero_i

    def in_triple(ci, s):
        r0 = wid * rays_per_w + ci * _CHUNK
        return ((z_hbm.at[pl.ds(r0, _CHUNK), :], z_v.at[s]),
                (w_hbm.at[0, pl.ds(r0, _CHUNK), :], w_v.at[s, 0]),
                (w_hbm.at[1, pl.ds(r0, _CHUNK), :], w_v.at[s, 1]))

    def out_pair(ci, s):
        r0 = wid * rays_per_w + ci * _CHUNK
        return ((out_v.at[s, 0], out_hbm.at[0, pl.ds(r0, _CHUNK), :]),
                (out_v.at[s, 1], out_hbm.at[1, pl.ds(r0, _CHUNK), :]))

    def issue_in(ci, s):
        for src, dst in in_triple(ci, s):
            pltpu.async_copy(src, dst, sin[s])

    def wait_in(ci, s):
        for src, dst in in_triple(ci, s):
            pltpu.make_async_copy(src, dst, sin[s]).wait()

    def issue_out(ci, s):
        for src, dst in out_pair(ci, s):
            pltpu.async_copy(src, dst, sout[s])

    def wait_out(ci, s):
        for src, dst in out_pair(ci, s):
            pltpu.make_async_copy(src, dst, sout[s]).wait()

    def chunk_compute(ci, s):
        svec = jnp.full((_L,), s, I32)

        def do_ray(c, binsb, cdfb, hist, histA):
            cvec = jnp.full((_L,), 0, I32) + c
            for j in range(n_vec):
                z0 = z_v[s, c, pl.ds(_L * j, _L)]
                idx1 = iota + (_L * j + 1)
                if j == n_vec - 1:
                    idx1 = jnp.minimum(idx1, 127)
                z1 = plsc.load_gather(z_v, [svec, cvec, idx1])
                binsb[pl.ds(_L * j, _L)] = (z0 + z1) * F32(0.5)
            eps = eps_v[...]

            for b in range(_B):
                bvec = jnp.full((_L,), b, I32)
                cs_l, t_l = [], []
                for j in range(n_vec):
                    a = w_v[s, b, c, pl.ds(_L * j, _L)] + eps
                    if j == 0:
                        a = jnp.where(iota != 0, a, F32(0.0))
                    if j == n_vec - 1:
                        a = jnp.where(iota != _L - 1, a, F32(0.0))
                    cs = plsc.cumsum(a)
                    cs_l.append(cs)
                    t_l.append(jnp.max(cs))
                off = F32(0.0)
                offs = []
                for j in range(n_vec):
                    offs.append(off)
                    off = off + t_l[j]
                total = off
                rT127 = jnp.full((_L,), F32(127.0)) / total
                thrv = jnp.full((_L,), F32(1e-5)) * total
                for j in range(n_vec):
                    cv = cs_l[j] + offs[j]
                    cdfb[pl.ds(_L * j, _L)] = cv
                    m = cv * rT127
                    ti = m.astype(I32)
                    ti = jnp.where(ti.astype(F32) < m, ti + 1, ti)
                    cp = jnp.minimum(ti, 128)
                    if j == 0:
                        cp = jnp.where(iota != 0, cp, 128)
                    if j == n_vec - 1:
                        cp = jnp.where(iota != _L - 1, cp, 128)
                    plsc.addupdate_scatter(hist, [cp], ones_i)
                ih_l, it_l = [], []
                for j in range(n_vec):
                    hv = hist[pl.ds(_L * j, _L)]
                    hist[pl.ds(_L * j, _L)] = zero_i
                    csh = plsc.cumsum(hv)
                    ih_l.append(csh)
                    it_l.append(jnp.max(csh))
                ioff = 0
                ioffs = []
                for j in range(n_vec):
                    ioffs.append(ioff)
                    ioff = ioff + it_l[j]
                for j in range(n_vec):
                    ik = ih_l[j] + ioffs[j]
                    i1 = ik + 1
                    g0 = plsc.load_gather(cdfb, [ik])
                    g1 = plsc.load_gather(cdfb, [i1])
                    b0 = plsc.load_gather(binsb, [ik])
                    b1 = plsc.load_gather(binsb, [i1])
                    den = g1 - g0
                    den = jnp.where(den < thrv, F32(1.0), den)
                    u = (iota + (_L * j)).astype(F32) * F32(1.0 / 127.0)
                    sv = b0 + (u * total - g0) / den * (b1 - b0)
                    zi1 = plsc.load_gather(z_v, [svec, cvec, i1])
                    av = i1 + jnp.where(sv >= zi1, 1, 0)
                    plsc.addupdate_scatter(histA, [av], ones_i)
                    plsc.store_scatter(
                        out_v, [svec, bvec, cvec, iota + (_L * j) + av], sv)
                bh_l, bt_l = [], []
                for j in range(n_vec):
                    hv = histA[pl.ds(_L * j, _L)]
                    histA[pl.ds(_L * j, _L)] = zero_i
                    csb = plsc.cumsum(hv)
                    bh_l.append(csb)
                    bt_l.append(jnp.max(csb))
                boff = 0
                boffs = []
                for j in range(n_vec):
                    boffs.append(boff)
                    boff = boff + bt_l[j]
                for j in range(n_vec):
                    bk = bh_l[j] + boffs[j]
                    zc = z_v[s, c, pl.ds(_L * j, _L)]
                    plsc.store_scatter(
                        out_v, [svec, bvec, cvec, iota + (_L * j) + bk], zc)

        def ray_body(c, _):
            do_ray(c, binsb, cdfb, hist, histA)
            return 0

        lax.fori_loop(0, _CHUNK, ray_body, 0)

    issue_in(0, 0)

    def outer_body(i2, _):
        for s in range(2):
            ci = i2 * 2 + s
            wait_in(ci, s)

            @pl.when(ci + 1 < n_chunks)
            def _prefetch():
                issue_in(ci + 1, 1 - s)

            @pl.when(ci >= 2)
            def _drain():
                wait_out(ci - 2, s)

            chunk_compute(ci, s)
            issue_out(ci, s)
        return 0

    lax.fori_loop(0, n_chunks // 2, outer_body, 0)
    wait_out(n_chunks - 2, 0)
    wait_out(n_chunks - 1, 1)


def _sc_zall(z_vals, weights, eps_vec):
    mesh = plsc.VectorSubcoreMesh(core_axis_name="c", subcore_axis_name="s")
    f = pl.kernel(
        _sc_zall_body,
        out_type=jax.ShapeDtypeStruct((_B, _R, 2 * _NS), F32),
        mesh=mesh,
        compiler_params=pltpu.CompilerParams(needs_layout_passes=False),
        scratch_types=[
            pltpu.VMEM((2, _CHUNK, _S), F32),
            pltpu.VMEM((2, _B, _CHUNK, _S), F32),
            pltpu.VMEM((2, _B, _CHUNK, 2 * _NS), F32),
            pltpu.VMEM((_L,), F32),
            pltpu.VMEM((144,), F32),
            pltpu.VMEM((144,), F32),
            pltpu.VMEM((144,), I32),
            pltpu.VMEM((144,), I32),
            pltpu.SemaphoreType.DMA,
            pltpu.SemaphoreType.DMA,
            pltpu.SemaphoreType.DMA,
            pltpu.SemaphoreType.DMA,
        ],
    )
    return f(z_vals, weights, eps_vec)


_RBLK = 512


def _tc_pts_body(z_ref, o_ref, d_ref, out_ref):
    z = z_ref[0]
    o = o_ref[...]
    d = d_ref[...]
    for c in range(3):
        out_ref[0, c] = o[:, c:c + 1] + d[:, c:c + 1] * z


def _tc_pts(z_all, rays_o_f, rays_d_f):
    nrb = _R // _RBLK
    grid = (_B, nrb)
    out4 = pl.pallas_call(
        _tc_pts_body,
        out_shape=jax.ShapeDtypeStruct((_B, 3, _R, 2 * _NS), F32),
        grid=grid,
        in_specs=[
            pl.BlockSpec((1, _RBLK, 2 * _NS), lambda b, rb: (b, rb, 0)),
            pl.BlockSpec((_RBLK, 3), lambda b, rb: (b * nrb + rb, 0)),
            pl.BlockSpec((_RBLK, 3), lambda b, rb: (b * nrb + rb, 0)),
        ],
        out_specs=pl.BlockSpec((1, 3, _RBLK, 2 * _NS),
                               lambda b, rb: (b, 0, rb, 0)),
    )(z_all, rays_o_f, rays_d_f)
    return jnp.transpose(out4, (0, 2, 3, 1))


def kernel(rays_d, rays_o, z_vals, weights, is_deterministic):
    z_vals = z_vals.reshape(-1, z_vals.shape[-1])
    eps_vec = jnp.full((_L,), _EPS, F32) * is_deterministic.astype(F32)
    z_all = _sc_zall(z_vals, weights, eps_vec)
    pts = _tc_pts(z_all, rays_o.reshape(_B * _R, 3), rays_d.reshape(_B * _R, 3))
    return pts, z_all


# speedup vs baseline: 2.1426x; 1.0009x over previous
import functools

import jax
import jax.numpy as jnp
from jax import lax
from jax.experimental import pallas as pl
from jax.experimental.pallas import tpu as pltpu
from jax.experimental.pallas import tpu_sc as plsc

F32 = jnp.float32
I32 = jnp.int32

_B = 2
_R = 32768
_S = 128
_NS = 128
_EPS = 1e-5
_NW = 32
_CHUNK = 64
_L = 16


def _sc_zall_body(z_hbm, w_hbm, eps_hbm, out_hbm,
                  z_v, w_v, out_v, eps_v, binsb, cdfb, hist, histA,
                  cdfb2, hist2, histA2,
                  sin0, sin1, sout0, sout1):
    wid = lax.axis_index("s") * 2 + lax.axis_index("c")
    rays_per_w = _R // _NW
    n_chunks = rays_per_w // _CHUNK
    sin = (sin0, sin1)
    sout = (sout0, sout1)

    pltpu.sync_copy(eps_hbm, eps_v)

    iota = lax.iota(I32, _L)
    ones_i = jnp.ones((_L,), I32)
    zero_i = jnp.zeros((_L,), I32)
    n_vec = _S // _L
    for jj in range(9):
        hist[pl.ds(_L * jj, _L)] = zero_i
        histA[pl.ds(_L * jj, _L)] = zero_i
        hist2[pl.ds(_L * jj, _L)] = zero_i
        histA2[pl.ds(_L * jj, _L)] = zero_i

    def in_triple(ci, s):
        r0 = wid * rays_per_w + ci * _CHUNK
        return ((z_hbm.at[pl.ds(r0, _CHUNK), :], z_v.at[s]),
                (w_hbm.at[0, pl.ds(r0, _CHUNK), :], w_v.at[s, 0]),
                (w_hbm.at[1, pl.ds(r0, _CHUNK), :], w_v.at[s, 1]))

    def out_pair(ci, s):
        r0 = wid * rays_per_w + ci * _CHUNK
        return ((out_v.at[s, 0], out_hbm.at[0, pl.ds(r0, _CHUNK), :]),
                (out_v.at[s, 1], out_hbm.at[1, pl.ds(r0, _CHUNK), :]))

    def issue_in(ci, s):
        for src, dst in in_triple(ci, s):
            pltpu.async_copy(src, dst, sin[s])

    def wait_in(ci, s):
        for src, dst in in_triple(ci, s):
            pltpu.make_async_copy(src, dst, sin[s]).wait()

    def issue_out(ci, s):
        for src, dst in out_pair(ci, s):
            pltpu.async_copy(src, dst, sout[s])

    def wait_out(ci, s):
        for src, dst in out_pair(ci, s):
            pltpu.make_async_copy(src, dst, sout[s]).wait()

    def chunk_compute(ci, s):
        svec = jnp.full((_L,), s, I32)

        def do_ray(c, binsb, cdfb1_, hist1_, histA1_):
            cvec = jnp.full((_L,), 0, I32) + c
            for j in range(n_vec):
                z0 = z_v[s, c, pl.ds(_L * j, _L)]
                idx1 = iota + (_L * j + 1)
                if j == n_vec - 1:
                    idx1 = jnp.minimum(idx1, 127)
                z1 = plsc.load_gather(z_v, [svec, cvec, idx1])
                binsb[pl.ds(_L * j, _L)] = (z0 + z1) * F32(0.5)
            eps = eps_v[...]

            for b in range(_B):
                bvec = jnp.full((_L,), b, I32)
                cdfb = (cdfb1_, cdfb2)[b]
                hist = (hist1_, hist2)[b]
                histA = (histA1_, histA2)[b]
                cs_l, t_l = [], []
                for j in range(n_vec):
                    a = w_v[s, b, c, pl.ds(_L * j, _L)] + eps
                    if j == 0:
                        a = jnp.where(iota != 0, a, F32(0.0))
                    if j == n_vec - 1:
                        a = jnp.where(iota != _L - 1, a, F32(0.0))
                    cs = plsc.cumsum(a)
                    cs_l.append(cs)
                    t_l.append(jnp.max(cs))
                off = F32(0.0)
                offs = []
                for j in range(n_vec):
                    offs.append(off)
                    off = off + t_l[j]
                total = off
                rT127 = jnp.full((_L,), F32(127.0)) / total
                thrv = jnp.full((_L,), F32(1e-5)) * total
                for j in range(n_vec):
                    cv = cs_l[j] + offs[j]
                    cdfb[pl.ds(_L * j, _L)] = cv
                    m = cv * rT127
                    ti = m.astype(I32)
                    ti = jnp.where(ti.astype(F32) < m, ti + 1, ti)
                    cp = jnp.minimum(ti, 128)
                    if j == 0:
                        cp = jnp.where(iota != 0, cp, 128)
                    if j == n_vec - 1:
                        cp = jnp.where(iota != _L - 1, cp, 128)
                    plsc.addupdate_scatter(hist, [cp], ones_i)
                ih_l, it_l = [], []
                for j in range(n_vec):
                    hv = hist[pl.ds(_L * j, _L)]
                    hist[pl.ds(_L * j, _L)] = zero_i
                    csh = plsc.cumsum(hv)
                    ih_l.append(csh)
                    it_l.append(jnp.max(csh))
                ioff = 0
                ioffs = []
                for j in range(n_vec):
                    ioffs.append(ioff)
                    ioff = ioff + it_l[j]
                for j in range(n_vec):
                    ik = ih_l[j] + ioffs[j]
                    i1 = ik + 1
                    g0 = plsc.load_gather(cdfb, [ik])
                    g1 = plsc.load_gather(cdfb, [i1])
                    b0 = plsc.load_gather(binsb, [ik])
                    b1 = plsc.load_gather(binsb, [i1])
                    den = g1 - g0
                    den = jnp.where(den < thrv, F32(1.0), den)
                    u = (iota + (_L * j)).astype(F32) * F32(1.0 / 127.0)
                    sv = b0 + (u * total - g0) / den * (b1 - b0)
                    zi1 = plsc.load_gather(z_v, [svec, cvec, i1])
                    av = i1 + jnp.where(sv >= zi1, 1, 0)
                    plsc.addupdate_scatter(histA, [av], ones_i)
                    plsc.store_scatter(
                        out_v, [svec, bvec, cvec, iota + (_L * j) + av], sv)
                bh_l, bt_l = [], []
                for j in range(n_vec):
                    hv = histA[pl.ds(_L * j, _L)]
                    histA[pl.ds(_L * j, _L)] = zero_i
                    csb = plsc.cumsum(hv)
                    bh_l.append(csb)
                    bt_l.append(jnp.max(csb))
                boff = 0
                boffs = []
                for j in range(n_vec):
                    boffs.append(boff)
                    boff = boff + bt_l[j]
                for j in range(n_vec):
                    bk = bh_l[j] + boffs[j]
                    zc = z_v[s, c, pl.ds(_L * j, _L)]
                    plsc.store_scatter(
                        out_v, [svec, bvec, cvec, iota + (_L * j) + bk], zc)

        def ray_body(c, _):
            do_ray(c, binsb, cdfb, hist, histA)
            return 0

        lax.fori_loop(0, _CHUNK, ray_body, 0)

    issue_in(0, 0)

    def outer_body(i2, _):
        for s in range(2):
            ci = i2 * 2 + s
            wait_in(ci, s)

            @pl.when(ci + 1 < n_chunks)
            def _prefetch():
                issue_in(ci + 1, 1 - s)

            @pl.when(ci >= 2)
            def _drain():
                wait_out(ci - 2, s)

            chunk_compute(ci, s)
            issue_out(ci, s)
        return 0

    lax.fori_loop(0, n_chunks // 2, outer_body, 0)
    wait_out(n_chunks - 2, 0)
    wait_out(n_chunks - 1, 1)


def _sc_zall(z_vals, weights, eps_vec):
    mesh = plsc.VectorSubcoreMesh(core_axis_name="c", subcore_axis_name="s")
    f = pl.kernel(
        _sc_zall_body,
        out_type=jax.ShapeDtypeStruct((_B, _R, 2 * _NS), F32),
        mesh=mesh,
        compiler_params=pltpu.CompilerParams(needs_layout_passes=False),
        scratch_types=[
            pltpu.VMEM((2, _CHUNK, _S), F32),
            pltpu.VMEM((2, _B, _CHUNK, _S), F32),
            pltpu.VMEM((2, _B, _CHUNK, 2 * _NS), F32),
            pltpu.VMEM((_L,), F32),
            pltpu.VMEM((144,), F32),
            pltpu.VMEM((144,), F32),
            pltpu.VMEM((144,), I32),
            pltpu.VMEM((144,), I32),
            pltpu.VMEM((144,), F32),
            pltpu.VMEM((144,), I32),
            pltpu.VMEM((144,), I32),
            pltpu.SemaphoreType.DMA,
            pltpu.SemaphoreType.DMA,
            pltpu.SemaphoreType.DMA,
            pltpu.SemaphoreType.DMA,
        ],
    )
    return f(z_vals, weights, eps_vec)


_RBLK = 512


def _tc_pts_body(z_ref, o_ref, d_ref, out_ref):
    z = z_ref[0]
    o = o_ref[...]
    d = d_ref[...]
    for c in range(3):
        out_ref[0, c] = o[:, c:c + 1] + d[:, c:c + 1] * z


def _tc_pts(z_all, rays_o_f, rays_d_f):
    nrb = _R // _RBLK
    grid = (_B, nrb)
    out4 = pl.pallas_call(
        _tc_pts_body,
        out_shape=jax.ShapeDtypeStruct((_B, 3, _R, 2 * _NS), F32),
        grid=grid,
        in_specs=[
            pl.BlockSpec((1, _RBLK, 2 * _NS), lambda b, rb: (b, rb, 0)),
            pl.BlockSpec((_RBLK, 3), lambda b, rb: (b * nrb + rb, 0)),
            pl.BlockSpec((_RBLK, 3), lambda b, rb: (b * nrb + rb, 0)),
        ],
        out_specs=pl.BlockSpec((1, 3, _RBLK, 2 * _NS),
                               lambda b, rb: (b, 0, rb, 0)),
    )(z_all, rays_o_f, rays_d_f)
    return jnp.transpose(out4, (0, 2, 3, 1))


def kernel(rays_d, rays_o, z_vals, weights, is_deterministic):
    z_vals = z_vals.reshape(-1, z_vals.shape[-1])
    eps_vec = jnp.full((_L,), _EPS, F32) * is_deterministic.astype(F32)
    z_all = _sc_zall(z_vals, weights, eps_vec)
    pts = _tc_pts(z_all, rays_o.reshape(_B * _R, 3), rays_d.reshape(_B * _R, 3))
    return pts, z_all
